# baseline (speedup 1.0000x reference)
import functools

import jax
import jax.numpy as jnp
from jax import lax
from jax.experimental import pallas as pl
from jax.experimental.pallas import tpu as pltpu
from jax.experimental.pallas import tpu_sc as plsc

_U = 50000
_I = 50000
_E = 800000
_D = 64
_H = 128

_NC = 2
_NS = 16
_CHUNK = 128
_GRP = 512
_NCH = _GRP // _CHUNK
_GROUPS = 98
_EDGES_PER_TILE = _GRP * _GROUPS
_E_PAD = _EDGES_PER_TILE * _NS
_CHUNK_ROWS = _E_PAD // _CHUNK
_CHUNKS_PER_TILE = _CHUNK_ROWS // _NS
_ROWS_PER_TILE = 3136
_N_ACC = _ROWS_PER_TILE * _NS
_BLK = 2000


def _sc_segment_sum(table, gidx, dst, w, zeros):
  n_dst = zeros.shape[0]
  mesh = plsc.VectorSubcoreMesh(core_axis_name="c", subcore_axis_name="s")

  @functools.partial(
      pl.kernel,
      out_type=jax.ShapeDtypeStruct((_NC, n_dst, 32), jnp.float32),
      mesh=mesh,
      scratch_types=[
          pltpu.VMEM((_NCH, _CHUNK), jnp.int32),
          pltpu.VMEM((_NCH, _CHUNK), jnp.int32),
          pltpu.VMEM((_GRP,), jnp.float32),
          pltpu.VMEM((_GRP, 16), jnp.int32),
          pltpu.VMEM((_GRP, 32), jnp.float32),
          pltpu.VMEM_SHARED((n_dst, 32), jnp.float32),
          pltpu.SemaphoreType.DMA,
      ],
      compiler_params=pltpu.CompilerParams(use_tc_tiling_on_sc=False,
                                           needs_layout_passes=False),
  )
  def body(table_h, gidx_h, dst_h, w_h, zeros_h, out_h,
           idxb, dstb, wb, gb, sb, acc, sem):
    c = lax.axis_index("c")
    s = lax.axis_index("s")
    slab = pl.ds(s * _ROWS_PER_TILE, _ROWS_PER_TILE)
    pltpu.sync_copy(zeros_h.at[slab], acc.at[slab])
    plsc.subcore_barrier()

    def grp(g, carry):
      r0 = s * _CHUNKS_PER_TILE + g * _NCH
      pltpu.sync_copy(gidx_h.at[c, pl.ds(r0, _NCH)], idxb)
      pltpu.sync_copy(dst_h.at[pl.ds(r0, _NCH)], dstb)
      pltpu.sync_copy(w_h.at[pl.ds(r0 * _CHUNK, _GRP)], wb)
      cps = [
          pltpu.async_copy(table_h.at[idxb.at[j]],
                           gb.at[pl.ds(j * _CHUNK, _CHUNK)], sem)
          for j in range(_NCH)
      ]
      for cp in cps:
        cp.wait()

      def scale(k16, acc_carry):
        wv16 = wb[pl.ds(k16 * 16, 16)]
        for j in range(16):
          e = k16 * 16 + j
          wv = wv16[j]
          xi = gb[e, :]
          lo = plsc.bitcast(xi << 16, jnp.float32)
          hi = plsc.bitcast(xi & -65536, jnp.float32)
          sb[e, pl.ds(0, 16)] = lo * wv
          sb[e, pl.ds(16, 16)] = hi * wv
        return acc_carry

      lax.fori_loop(0, _GRP // 16, scale, 0)
      for j in range(_NCH):
        pltpu.sync_copy(sb.at[pl.ds(j * _CHUNK, _CHUNK)],
                        acc.at[dstb.at[j]], add=True)
      return carry

    lax.fori_loop(0, _GROUPS, grp, 0)
    plsc.subcore_barrier()
    pltpu.sync_copy(acc.at[slab], out_h.at[c, slab])

  return body(table, gidx, dst, w, zeros)


def _pack_table(split):
  n = split.shape[1]
  pairs = jnp.stack([split[..., :16], split[..., 16:]], axis=-1)
  xb = pairs.astype(jnp.bfloat16)
  xi = jax.lax.bitcast_convert_type(xb, jnp.int32)
  return xi.reshape(2 * n, 16)


def _hyper_and_split(user_emb, item_emb, user_mat, item_mat):
  def body(u_ref, i_ref, mu_ref, mi_ref, hu_ref, hi_ref, us_ref, is_ref):
    u = u_ref[...]
    it = i_ref[...]
    hu_ref[...] = jnp.dot(u, mu_ref[...], preferred_element_type=jnp.float32)
    hi_ref[...] = jnp.dot(it, mi_ref[...], preferred_element_type=jnp.float32)
    us_ref[...] = jnp.stack([u[:, :32], u[:, 32:]], axis=0)
    is_ref[...] = jnp.stack([it[:, :32], it[:, 32:]], axis=0)

  grid = (_U // _BLK,)
  return pl.pallas_call(
      body,
      grid=grid,
      in_specs=[
          pl.BlockSpec((_BLK, _D), lambda i: (i, 0)),
          pl.BlockSpec((_BLK, _D), lambda i: (i, 0)),
          pl.BlockSpec((_D, _H), lambda i: (0, 0)),
          pl.BlockSpec((_D, _H), lambda i: (0, 0)),
      ],
      out_specs=[
          pl.BlockSpec((_BLK, _H), lambda i: (i, 0)),
          pl.BlockSpec((_BLK, _H), lambda i: (i, 0)),
          pl.BlockSpec((2, _BLK, 32), lambda i: (0, i, 0)),
          pl.BlockSpec((2, _BLK, 32), lambda i: (0, i, 0)),
      ],
      out_shape=[
          jax.ShapeDtypeStruct((_U, _H), jnp.float32),
          jax.ShapeDtypeStruct((_I, _H), jnp.float32),
          jax.ShapeDtypeStruct((2, _U, 32), jnp.float32),
          jax.ShapeDtypeStruct((2, _I, 32), jnp.float32),
      ],
  )(user_emb, item_emb, user_mat, item_mat)


def _he_reduce(hyper_u, hyper_i, u_split, i_split):
  def body(hu_ref, us_ref, hi_ref, is_ref, heu_ref, hei_ref):
    step = pl.program_id(0)
    u = jnp.concatenate([us_ref[0], us_ref[1]], axis=1)
    it = jnp.concatenate([is_ref[0], is_ref[1]], axis=1)
    dn = (((0,), (0,)), ((), ()))
    pu = lax.dot_general(hu_ref[...], u, dn,
                         preferred_element_type=jnp.float32)
    pi = lax.dot_general(hi_ref[...], it, dn,
                         preferred_element_type=jnp.float32)

    @pl.when(step == 0)
    def _():
      heu_ref[...] = pu
      hei_ref[...] = pi

    @pl.when(step != 0)
    def _():
      heu_ref[...] += pu
      hei_ref[...] += pi

  grid = (_U // _BLK,)
  return pl.pallas_call(
      body,
      grid=grid,
      in_specs=[
          pl.BlockSpec((_BLK, _H), lambda i: (i, 0)),
          pl.BlockSpec((2, _BLK, 32), lambda i: (0, i, 0)),
          pl.BlockSpec((_BLK, _H), lambda i: (i, 0)),
          pl.BlockSpec((2, _BLK, 32), lambda i: (0, i, 0)),
      ],
      out_specs=[
          pl.BlockSpec((_H, _D), lambda i: (0, 0)),
          pl.BlockSpec((_H, _D), lambda i: (0, 0)),
      ],
      out_shape=[
          jax.ShapeDtypeStruct((_H, _D), jnp.float32),
          jax.ShapeDtypeStruct((_H, _D), jnp.float32),
      ],
  )(hyper_u, u_split, hyper_i, i_split)


def _combine(hyper_u, he_u, raw_u, acc_u, hyper_i, he_i, raw_i, acc_i):
  def body(hu_ref, heu_ref, ru_ref, au_ref, hi_ref, hei_ref, ri_ref, ai_ref,
           lu_ref, gu_ref, nu_ref, aou_ref, li_ref, gi_ref, ni_ref, aoi_ref):
    def one(h_ref, he_ref, r_ref, a_ref, l_ref, g_ref, n_ref, ao_ref):
      g = jnp.dot(h_ref[...], he_ref[...],
                  preferred_element_type=jnp.float32)
      raw = jnp.concatenate([r_ref[0], r_ref[1]], axis=1)
      loc = jnp.where(raw >= 0, raw, 0.5 * raw)
      nxt = loc + g
      l_ref[...] = loc
      g_ref[...] = g
      n_ref[...] = jnp.stack([nxt[:, :32], nxt[:, 32:]], axis=0)
      ao_ref[...] = a_ref[...] + nxt

    one(hu_ref, heu_ref, ru_ref, au_ref, lu_ref, gu_ref, nu_ref, aou_ref)
    one(hi_ref, hei_ref, ri_ref, ai_ref, li_ref, gi_ref, ni_ref, aoi_ref)

  grid = (_U // _BLK,)
  row_spec = pl.BlockSpec((_BLK, _D), lambda i: (i, 0))
  split_spec = pl.BlockSpec((2, _BLK, 32), lambda i: (0, i, 0))
  he_spec = pl.BlockSpec((_H, _D), lambda i: (0, 0))
  h_spec = pl.BlockSpec((_BLK, _H), lambda i: (i, 0))
  row_shape = jax.ShapeDtypeStruct((_U, _D), jnp.float32)
  split_shape = jax.ShapeDtypeStruct((2, _U, 32), jnp.float32)
  return pl.pallas_call(
      body,
      grid=grid,
      in_specs=[h_spec, he_spec, split_spec, row_spec] * 2,
      out_specs=[row_spec, row_spec, split_spec, row_spec] * 2,
      out_shape=[row_shape, row_shape, split_shape, row_shape] * 2,
  )(hyper_u, he_u, raw_u, acc_u, hyper_i, he_i, raw_i, acc_i)


def kernel(user_emb, item_emb, user_mat, item_mat, edge_index, edge_weight):
  row = edge_index[0].astype(jnp.int32)
  col = edge_index[1].astype(jnp.int32)
  pad = _E_PAD - _E
  rowp = jnp.pad(row, (0, pad))
  colp = jnp.pad(col, (0, pad))
  wp = jnp.pad(edge_weight.astype(jnp.float32), (0, pad))
  gidx_u = jnp.stack([colp, colp + _I]).reshape(_NC, _CHUNK_ROWS, _CHUNK)
  gidx_i = jnp.stack([rowp, rowp + _U]).reshape(_NC, _CHUNK_ROWS, _CHUNK)
  dst_u = rowp.reshape(_CHUNK_ROWS, _CHUNK)
  dst_i = colp.reshape(_CHUNK_ROWS, _CHUNK)
  zeros = jnp.zeros((_N_ACC, 32), jnp.float32)

  hyper_u, hyper_i, u_split, i_split = _hyper_and_split(
      user_emb, item_emb, user_mat, item_mat)

  acc_u, acc_i = user_emb, item_emb
  locals_u, locals_i, globals_u, globals_i = [], [], [], []
  for _ in range(2):
    i_tab = _pack_table(i_split)
    u_tab = _pack_table(u_split)
    raw_u = _sc_segment_sum(i_tab, gidx_u, dst_u, wp, zeros)
    raw_i = _sc_segment_sum(u_tab, gidx_i, dst_i, wp, zeros)
    he_u, he_i = _he_reduce(hyper_u, hyper_i, u_split, i_split)
    (local_u, g_u, u_split, acc_u,
     local_i, g_i, i_split, acc_i) = _combine(
         hyper_u, he_u, raw_u, acc_u, hyper_i, he_i, raw_i, acc_i)
    locals_u.append(local_u)
    locals_i.append(local_i)
    globals_u.append(g_u)
    globals_i.append(g_i)

  return (acc_u, acc_i, *locals_u, *locals_i, *globals_u, *globals_i)


# speedup vs baseline: 1.4640x; 1.4640x over previous
import functools

import jax
import jax.numpy as jnp
from jax import lax
from jax.experimental import pallas as pl
from jax.experimental.pallas import tpu as pltpu
from jax.experimental.pallas import tpu_sc as plsc

_U = 50000
_I = 50000
_E = 800000
_D = 64
_H = 128

_NC = 2
_NS = 16
_CHUNK = 128
_GRP = 512
_NCH = _GRP // _CHUNK
_GROUPS = 98
_EDGES_PER_TILE = _GRP * _GROUPS
_E_PAD = _EDGES_PER_TILE * _NS
_CHUNK_ROWS = _E_PAD // _CHUNK
_CHUNKS_PER_TILE = _CHUNK_ROWS // _NS
_ROWS_PER_TILE = 3136
_N_ACC = _ROWS_PER_TILE * _NS
_BLK = 2000


def _sc_segment_sum(table, gidx, dst, w, zeros):
  n_dst = zeros.shape[0]
  mesh = plsc.VectorSubcoreMesh(core_axis_name="c", subcore_axis_name="s")

  @functools.partial(
      pl.kernel,
      out_type=jax.ShapeDtypeStruct((_NC, n_dst, 32), jnp.float32),
      mesh=mesh,
      scratch_types=[
          pltpu.VMEM((_NCH, _CHUNK), jnp.int32),
          pltpu.VMEM((_NCH, _CHUNK), jnp.int32),
          pltpu.VMEM((_GRP,), jnp.float32),
          pltpu.VMEM((_GRP, 32), jnp.float32),
          pltpu.VMEM_SHARED((n_dst, 32), jnp.float32),
          pltpu.SemaphoreType.DMA,
      ],
      compiler_params=pltpu.CompilerParams(use_tc_tiling_on_sc=False),
  )
  def body(table_h, gidx_h, dst_h, w_h, zeros_h, out_h,
           idxb, dstb, wb, gb, acc, sem):
    c = lax.axis_index("c")
    s = lax.axis_index("s")
    slab = pl.ds(s * _ROWS_PER_TILE, _ROWS_PER_TILE)
    pltpu.sync_copy(zeros_h.at[slab], acc.at[slab])
    plsc.subcore_barrier()

    def grp(g, carry):
      r0 = s * _CHUNKS_PER_TILE + g * _NCH
      pltpu.sync_copy(gidx_h.at[c, pl.ds(r0, _NCH)], idxb)
      pltpu.sync_copy(dst_h.at[pl.ds(r0, _NCH)], dstb)
      pltpu.sync_copy(w_h.at[pl.ds(r0 * _CHUNK, _GRP)], wb)
      cps = [
          pltpu.async_copy(table_h.at[idxb.at[j]],
                           gb.at[pl.ds(j * _CHUNK, _CHUNK)], sem)
          for j in range(_NCH)
      ]
      for cp in cps:
        cp.wait()

      for k16 in range(_GRP // 16):
        wv16 = wb[pl.ds(k16 * 16, 16)]
        for j in range(16):
          e = k16 * 16 + j
          wv = wv16[j]
          gb[e, pl.ds(0, 16)] = gb[e, pl.ds(0, 16)] * wv
          gb[e, pl.ds(16, 16)] = gb[e, pl.ds(16, 16)] * wv
      for j in range(_NCH):
        pltpu.sync_copy(gb.at[pl.ds(j * _CHUNK, _CHUNK)],
                        acc.at[dstb.at[j]], add=True)
      return carry

    lax.fori_loop(0, _GROUPS, grp, 0)
    plsc.subcore_barrier()
    pltpu.sync_copy(acc.at[slab], out_h.at[c, slab])

  return body(table, gidx, dst, w, zeros)


def _hyper_and_split(user_emb, item_emb, user_mat, item_mat):
  def body(u_ref, i_ref, mu_ref, mi_ref, hu_ref, hi_ref, us_ref, is_ref):
    u = u_ref[...]
    it = i_ref[...]
    hu_ref[...] = jnp.dot(u, mu_ref[...], preferred_element_type=jnp.float32)
    hi_ref[...] = jnp.dot(it, mi_ref[...], preferred_element_type=jnp.float32)
    us_ref[...] = jnp.stack([u[:, :32], u[:, 32:]], axis=0)
    is_ref[...] = jnp.stack([it[:, :32], it[:, 32:]], axis=0)

  grid = (_U // _BLK,)
  return pl.pallas_call(
      body,
      grid=grid,
      in_specs=[
          pl.BlockSpec((_BLK, _D), lambda i: (i, 0)),
          pl.BlockSpec((_BLK, _D), lambda i: (i, 0)),
          pl.BlockSpec((_D, _H), lambda i: (0, 0)),
          pl.BlockSpec((_D, _H), lambda i: (0, 0)),
      ],
      out_specs=[
          pl.BlockSpec((_BLK, _H), lambda i: (i, 0)),
          pl.BlockSpec((_BLK, _H), lambda i: (i, 0)),
          pl.BlockSpec((2, _BLK, 32), lambda i: (0, i, 0)),
          pl.BlockSpec((2, _BLK, 32), lambda i: (0, i, 0)),
      ],
      out_shape=[
          jax.ShapeDtypeStruct((_U, _H), jnp.float32),
          jax.ShapeDtypeStruct((_I, _H), jnp.float32),
          jax.ShapeDtypeStruct((2, _U, 32), jnp.float32),
          jax.ShapeDtypeStruct((2, _I, 32), jnp.float32),
      ],
  )(user_emb, item_emb, user_mat, item_mat)


def _he_reduce(hyper_u, hyper_i, u_split, i_split):
  def body(hu_ref, us_ref, hi_ref, is_ref, heu_ref, hei_ref):
    step = pl.program_id(0)
    u = jnp.concatenate([us_ref[0], us_ref[1]], axis=1)
    it = jnp.concatenate([is_ref[0], is_ref[1]], axis=1)
    dn = (((0,), (0,)), ((), ()))
    pu = lax.dot_general(hu_ref[...], u, dn,
                         preferred_element_type=jnp.float32)
    pi = lax.dot_general(hi_ref[...], it, dn,
                         preferred_element_type=jnp.float32)

    @pl.when(step == 0)
    def _():
      heu_ref[...] = pu
      hei_ref[...] = pi

    @pl.when(step != 0)
    def _():
      heu_ref[...] += pu
      hei_ref[...] += pi

  grid = (_U // _BLK,)
  return pl.pallas_call(
      body,
      grid=grid,
      in_specs=[
          pl.BlockSpec((_BLK, _H), lambda i: (i, 0)),
          pl.BlockSpec((2, _BLK, 32), lambda i: (0, i, 0)),
          pl.BlockSpec((_BLK, _H), lambda i: (i, 0)),
          pl.BlockSpec((2, _BLK, 32), lambda i: (0, i, 0)),
      ],
      out_specs=[
          pl.BlockSpec((_H, _D), lambda i: (0, 0)),
          pl.BlockSpec((_H, _D), lambda i: (0, 0)),
      ],
      out_shape=[
          jax.ShapeDtypeStruct((_H, _D), jnp.float32),
          jax.ShapeDtypeStruct((_H, _D), jnp.float32),
      ],
  )(hyper_u, u_split, hyper_i, i_split)


def _combine(hyper_u, he_u, raw_u, acc_u, hyper_i, he_i, raw_i, acc_i):
  def body(hu_ref, heu_ref, ru_ref, au_ref, hi_ref, hei_ref, ri_ref, ai_ref,
           lu_ref, gu_ref, nu_ref, aou_ref, li_ref, gi_ref, ni_ref, aoi_ref):
    def one(h_ref, he_ref, r_ref, a_ref, l_ref, g_ref, n_ref, ao_ref):
      g = jnp.dot(h_ref[...], he_ref[...],
                  preferred_element_type=jnp.float32)
      raw = jnp.concatenate([r_ref[0], r_ref[1]], axis=1)
      loc = jnp.where(raw >= 0, raw, 0.5 * raw)
      nxt = loc + g
      l_ref[...] = loc
      g_ref[...] = g
      n_ref[...] = jnp.stack([nxt[:, :32], nxt[:, 32:]], axis=0)
      ao_ref[...] = a_ref[...] + nxt

    one(hu_ref, heu_ref, ru_ref, au_ref, lu_ref, gu_ref, nu_ref, aou_ref)
    one(hi_ref, hei_ref, ri_ref, ai_ref, li_ref, gi_ref, ni_ref, aoi_ref)

  grid = (_U // _BLK,)
  row_spec = pl.BlockSpec((_BLK, _D), lambda i: (i, 0))
  split_spec = pl.BlockSpec((2, _BLK, 32), lambda i: (0, i, 0))
  he_spec = pl.BlockSpec((_H, _D), lambda i: (0, 0))
  h_spec = pl.BlockSpec((_BLK, _H), lambda i: (i, 0))
  row_shape = jax.ShapeDtypeStruct((_U, _D), jnp.float32)
  split_shape = jax.ShapeDtypeStruct((2, _U, 32), jnp.float32)
  return pl.pallas_call(
      body,
      grid=grid,
      in_specs=[h_spec, he_spec, split_spec, row_spec] * 2,
      out_specs=[row_spec, row_spec, split_spec, row_spec] * 2,
      out_shape=[row_shape, row_shape, split_shape, row_shape] * 2,
  )(hyper_u, he_u, raw_u, acc_u, hyper_i, he_i, raw_i, acc_i)


def kernel(user_emb, item_emb, user_mat, item_mat, edge_index, edge_weight):
  row = edge_index[0].astype(jnp.int32)
  col = edge_index[1].astype(jnp.int32)
  pad = _E_PAD - _E
  rowp = jnp.pad(row, (0, pad))
  colp = jnp.pad(col, (0, pad))
  wp = jnp.pad(edge_weight.astype(jnp.float32), (0, pad))
  gidx_u = jnp.stack([colp, colp + _I]).reshape(_NC, _CHUNK_ROWS, _CHUNK)
  gidx_i = jnp.stack([rowp, rowp + _U]).reshape(_NC, _CHUNK_ROWS, _CHUNK)
  dst_u = rowp.reshape(_CHUNK_ROWS, _CHUNK)
  dst_i = colp.reshape(_CHUNK_ROWS, _CHUNK)
  zeros = jnp.zeros((_N_ACC, 32), jnp.float32)

  hyper_u, hyper_i, u_split, i_split = _hyper_and_split(
      user_emb, item_emb, user_mat, item_mat)

  acc_u, acc_i = user_emb, item_emb
  locals_u, locals_i, globals_u, globals_i = [], [], [], []
  for _ in range(2):
    i_tab = i_split.reshape(_NC * _I, 32)
    u_tab = u_split.reshape(_NC * _U, 32)
    raw_u = _sc_segment_sum(i_tab, gidx_u, dst_u, wp, zeros)
    raw_i = _sc_segment_sum(u_tab, gidx_i, dst_i, wp, zeros)
    he_u, he_i = _he_reduce(hyper_u, hyper_i, u_split, i_split)
    (local_u, g_u, u_split, acc_u,
     local_i, g_i, i_split, acc_i) = _combine(
         hyper_u, he_u, raw_u, acc_u, hyper_i, he_i, raw_i, acc_i)
    locals_u.append(local_u)
    locals_i.append(local_i)
    globals_u.append(g_u)
    globals_i.append(g_i)

  return (acc_u, acc_i, *locals_u, *locals_i, *globals_u, *globals_i)


# speedup vs baseline: 1.7457x; 1.1924x over previous
import functools

import jax
import jax.numpy as jnp
from jax import lax
from jax.experimental import pallas as pl
from jax.experimental.pallas import tpu as pltpu
from jax.experimental.pallas import tpu_sc as plsc

_U = 50000
_I = 50000
_E = 800000
_D = 64
_H = 128

_NC = 2
_NS = 16
_CHUNK = 128
_GRP = 512
_NCH = _GRP // _CHUNK
_GROUPS = 98
_EDGES_PER_TILE = _GRP * _GROUPS
_E_PAD = _EDGES_PER_TILE * _NS
_CHUNK_ROWS = _E_PAD // _CHUNK
_CHUNKS_PER_TILE = _CHUNK_ROWS // _NS
_ROWS_PER_TILE = 3136
_N_ACC = _ROWS_PER_TILE * _NS
_BLK = 2000


def _sc_segment_sum(table, gidx, dst, w, zeros):
  n_dst = zeros.shape[0]
  mesh = plsc.VectorSubcoreMesh(core_axis_name="c", subcore_axis_name="s")

  @functools.partial(
      pl.kernel,
      out_type=jax.ShapeDtypeStruct((_NC, n_dst, 32), jnp.float32),
      mesh=mesh,
      scratch_types=[
          pltpu.VMEM((_NCH, _CHUNK), jnp.int32),
          pltpu.VMEM((_NCH, _CHUNK), jnp.int32),
          pltpu.VMEM((_GRP,), jnp.float32),
          pltpu.VMEM((_GRP, 32), jnp.float32),
          pltpu.VMEM_SHARED((n_dst, 32), jnp.float32),
          pltpu.SemaphoreType.DMA,
          pltpu.SemaphoreType.DMA,
          pltpu.SemaphoreType.DMA,
      ],
      compiler_params=pltpu.CompilerParams(use_tc_tiling_on_sc=False),
  )
  def body(table_h, gidx_h, dst_h, w_h, zeros_h, out_h,
           idxb, dstb, wb, gb, acc, sem, isem, ssem):
    c = lax.axis_index("c")
    s = lax.axis_index("s")
    slab = pl.ds(s * _ROWS_PER_TILE, _ROWS_PER_TILE)
    pltpu.sync_copy(zeros_h.at[slab], acc.at[slab])
    plsc.subcore_barrier()

    def grp(g, carry):
      r0 = s * _CHUNKS_PER_TILE + g * _NCH
      ils = [
          pltpu.async_copy(gidx_h.at[c, pl.ds(r0, _NCH)], idxb, isem),
          pltpu.async_copy(dst_h.at[pl.ds(r0, _NCH)], dstb, isem),
          pltpu.async_copy(w_h.at[pl.ds(r0 * _CHUNK, _GRP)], wb, isem),
      ]
      for cp in ils:
        cp.wait()
      cps = [
          pltpu.async_copy(table_h.at[idxb.at[j]],
                           gb.at[pl.ds(j * _CHUNK, _CHUNK)], sem)
          for j in range(_NCH)
      ]
      for cp in cps:
        cp.wait()

      for k16 in range(_GRP // 16):
        wv16 = wb[pl.ds(k16 * 16, 16)]
        for j in range(16):
          e = k16 * 16 + j
          wv = wv16[j]
          gb[e, pl.ds(0, 16)] = gb[e, pl.ds(0, 16)] * wv
          gb[e, pl.ds(16, 16)] = gb[e, pl.ds(16, 16)] * wv
      scs = [
          pltpu.async_copy(gb.at[pl.ds(j * _CHUNK, _CHUNK)],
                           acc.at[dstb.at[j]], ssem, add=True)
          for j in range(_NCH)
      ]
      for cp in scs:
        cp.wait()
      return carry

    lax.fori_loop(0, _GROUPS, grp, 0)
    plsc.subcore_barrier()
    pltpu.sync_copy(acc.at[slab], out_h.at[c, slab])

  return body(table, gidx, dst, w, zeros)


def _hyper_and_split(user_emb, item_emb, user_mat, item_mat):
  def body(u_ref, i_ref, mu_ref, mi_ref, hu_ref, hi_ref, us_ref, is_ref):
    u = u_ref[...]
    it = i_ref[...]
    hu_ref[...] = jnp.dot(u, mu_ref[...], preferred_element_type=jnp.float32)
    hi_ref[...] = jnp.dot(it, mi_ref[...], preferred_element_type=jnp.float32)
    us_ref[...] = jnp.stack([u[:, :32], u[:, 32:]], axis=0)
    is_ref[...] = jnp.stack([it[:, :32], it[:, 32:]], axis=0)

  grid = (_U // _BLK,)
  return pl.pallas_call(
      body,
      grid=grid,
      in_specs=[
          pl.BlockSpec((_BLK, _D), lambda i: (i, 0)),
          pl.BlockSpec((_BLK, _D), lambda i: (i, 0)),
          pl.BlockSpec((_D, _H), lambda i: (0, 0)),
          pl.BlockSpec((_D, _H), lambda i: (0, 0)),
      ],
      out_specs=[
          pl.BlockSpec((_BLK, _H), lambda i: (i, 0)),
          pl.BlockSpec((_BLK, _H), lambda i: (i, 0)),
          pl.BlockSpec((2, _BLK, 32), lambda i: (0, i, 0)),
          pl.BlockSpec((2, _BLK, 32), lambda i: (0, i, 0)),
      ],
      out_shape=[
          jax.ShapeDtypeStruct((_U, _H), jnp.float32),
          jax.ShapeDtypeStruct((_I, _H), jnp.float32),
          jax.ShapeDtypeStruct((2, _U, 32), jnp.float32),
          jax.ShapeDtypeStruct((2, _I, 32), jnp.float32),
      ],
  )(user_emb, item_emb, user_mat, item_mat)


def _he_reduce(hyper_u, hyper_i, u_split, i_split):
  def body(hu_ref, us_ref, hi_ref, is_ref, heu_ref, hei_ref):
    step = pl.program_id(0)
    u = jnp.concatenate([us_ref[0], us_ref[1]], axis=1)
    it = jnp.concatenate([is_ref[0], is_ref[1]], axis=1)
    dn = (((0,), (0,)), ((), ()))
    pu = lax.dot_general(hu_ref[...], u, dn,
                         preferred_element_type=jnp.float32)
    pi = lax.dot_general(hi_ref[...], it, dn,
                         preferred_element_type=jnp.float32)

    @pl.when(step == 0)
    def _():
      heu_ref[...] = pu
      hei_ref[...] = pi

    @pl.when(step != 0)
    def _():
      heu_ref[...] += pu
      hei_ref[...] += pi

  grid = (_U // _BLK,)
  return pl.pallas_call(
      body,
      grid=grid,
      in_specs=[
          pl.BlockSpec((_BLK, _H), lambda i: (i, 0)),
          pl.BlockSpec((2, _BLK, 32), lambda i: (0, i, 0)),
          pl.BlockSpec((_BLK, _H), lambda i: (i, 0)),
          pl.BlockSpec((2, _BLK, 32), lambda i: (0, i, 0)),
      ],
      out_specs=[
          pl.BlockSpec((_H, _D), lambda i: (0, 0)),
          pl.BlockSpec((_H, _D), lambda i: (0, 0)),
      ],
      out_shape=[
          jax.ShapeDtypeStruct((_H, _D), jnp.float32),
          jax.ShapeDtypeStruct((_H, _D), jnp.float32),
      ],
  )(hyper_u, u_split, hyper_i, i_split)


def _combine(hyper_u, he_u, raw_u, acc_u, hyper_i, he_i, raw_i, acc_i):
  def body(hu_ref, heu_ref, ru_ref, au_ref, hi_ref, hei_ref, ri_ref, ai_ref,
           lu_ref, gu_ref, nu_ref, aou_ref, li_ref, gi_ref, ni_ref, aoi_ref):
    def one(h_ref, he_ref, r_ref, a_ref, l_ref, g_ref, n_ref, ao_ref):
      g = jnp.dot(h_ref[...], he_ref[...],
                  preferred_element_type=jnp.float32)
      raw = jnp.concatenate([r_ref[0], r_ref[1]], axis=1)
      loc = jnp.where(raw >= 0, raw, 0.5 * raw)
      nxt = loc + g
      l_ref[...] = loc
      g_ref[...] = g
      n_ref[...] = jnp.stack([nxt[:, :32], nxt[:, 32:]], axis=0)
      ao_ref[...] = a_ref[...] + nxt

    one(hu_ref, heu_ref, ru_ref, au_ref, lu_ref, gu_ref, nu_ref, aou_ref)
    one(hi_ref, hei_ref, ri_ref, ai_ref, li_ref, gi_ref, ni_ref, aoi_ref)

  grid = (_U // _BLK,)
  row_spec = pl.BlockSpec((_BLK, _D), lambda i: (i, 0))
  split_spec = pl.BlockSpec((2, _BLK, 32), lambda i: (0, i, 0))
  he_spec = pl.BlockSpec((_H, _D), lambda i: (0, 0))
  h_spec = pl.BlockSpec((_BLK, _H), lambda i: (i, 0))
  row_shape = jax.ShapeDtypeStruct((_U, _D), jnp.float32)
  split_shape = jax.ShapeDtypeStruct((2, _U, 32), jnp.float32)
  return pl.pallas_call(
      body,
      grid=grid,
      in_specs=[h_spec, he_spec, split_spec, row_spec] * 2,
      out_specs=[row_spec, row_spec, split_spec, row_spec] * 2,
      out_shape=[row_shape, row_shape, split_shape, row_shape] * 2,
  )(hyper_u, he_u, raw_u, acc_u, hyper_i, he_i, raw_i, acc_i)


def kernel(user_emb, item_emb, user_mat, item_mat, edge_index, edge_weight):
  row = edge_index[0].astype(jnp.int32)
  col = edge_index[1].astype(jnp.int32)
  pad = _E_PAD - _E
  rowp = jnp.pad(row, (0, pad))
  colp = jnp.pad(col, (0, pad))
  wp = jnp.pad(edge_weight.astype(jnp.float32), (0, pad))
  gidx_u = jnp.stack([colp, colp + _I]).reshape(_NC, _CHUNK_ROWS, _CHUNK)
  gidx_i = jnp.stack([rowp, rowp + _U]).reshape(_NC, _CHUNK_ROWS, _CHUNK)
  dst_u = rowp.reshape(_CHUNK_ROWS, _CHUNK)
  dst_i = colp.reshape(_CHUNK_ROWS, _CHUNK)
  zeros = jnp.zeros((_N_ACC, 32), jnp.float32)

  hyper_u, hyper_i, u_split, i_split = _hyper_and_split(
      user_emb, item_emb, user_mat, item_mat)

  acc_u, acc_i = user_emb, item_emb
  locals_u, locals_i, globals_u, globals_i = [], [], [], []
  for _ in range(2):
    i_tab = i_split.reshape(_NC * _I, 32)
    u_tab = u_split.reshape(_NC * _U, 32)
    raw_u = _sc_segment_sum(i_tab, gidx_u, dst_u, wp, zeros)
    raw_i = _sc_segment_sum(u_tab, gidx_i, dst_i, wp, zeros)
    he_u, he_i = _he_reduce(hyper_u, hyper_i, u_split, i_split)
    (local_u, g_u, u_split, acc_u,
     local_i, g_i, i_split, acc_i) = _combine(
         hyper_u, he_u, raw_u, acc_u, hyper_i, he_i, raw_i, acc_i)
    locals_u.append(local_u)
    locals_i.append(local_i)
    globals_u.append(g_u)
    globals_i.append(g_i)

  return (acc_u, acc_i, *locals_u, *locals_i, *globals_u, *globals_i)


# speedup vs baseline: 1.8489x; 1.0592x over previous
import functools

import jax
import jax.numpy as jnp
from jax import lax
from jax.experimental import pallas as pl
from jax.experimental.pallas import tpu as pltpu
from jax.experimental.pallas import tpu_sc as plsc

_U = 50000
_I = 50000
_E = 800000
_D = 64
_H = 128

_NC = 2
_NS = 16
_CHUNK = 128
_GRP = 512
_NCH = _GRP // _CHUNK
_GROUPS = 98
_EDGES_PER_TILE = _GRP * _GROUPS
_E_PAD = _EDGES_PER_TILE * _NS
_CHUNK_ROWS = _E_PAD // _CHUNK
_CHUNKS_PER_TILE = _CHUNK_ROWS // _NS
_ROWS_PER_TILE = 3136
_N_ACC = _ROWS_PER_TILE * _NS
_BLK = 2000


def _sc_segment_sum(table, gidx, dst, w, zeros):
  n_dst = zeros.shape[0]
  mesh = plsc.VectorSubcoreMesh(core_axis_name="c", subcore_axis_name="s")

  @functools.partial(
      pl.kernel,
      out_type=jax.ShapeDtypeStruct((_NC, n_dst, 32), jnp.float32),
      mesh=mesh,
      scratch_types=[
          pltpu.VMEM((_NCH, _CHUNK), jnp.int32),
          pltpu.VMEM((_NCH, _CHUNK), jnp.int32),
          pltpu.VMEM((_GRP,), jnp.float32),
          pltpu.VMEM((_GRP, 32), jnp.float32),
          pltpu.VMEM_SHARED((n_dst, 32), jnp.float32),
          pltpu.SemaphoreType.DMA,
          pltpu.SemaphoreType.DMA,
          pltpu.SemaphoreType.DMA,
          pltpu.SemaphoreType.DMA,
      ],
      compiler_params=pltpu.CompilerParams(use_tc_tiling_on_sc=False),
  )
  def body(table_h, gidx_h, dst_h, w_h, zeros_h, out_h,
           idxb, dstb, wb, gb, acc, sem, sem2, isem, ssem):
    c = lax.axis_index("c")
    s = lax.axis_index("s")
    slab = pl.ds(s * _ROWS_PER_TILE, _ROWS_PER_TILE)
    pltpu.sync_copy(zeros_h.at[slab], acc.at[slab])
    plsc.subcore_barrier()

    def grp(g, carry):
      r0 = s * _CHUNKS_PER_TILE + g * _NCH
      ils = [
          pltpu.async_copy(gidx_h.at[c, pl.ds(r0, _NCH)], idxb, isem),
          pltpu.async_copy(dst_h.at[pl.ds(r0, _NCH)], dstb, isem),
          pltpu.async_copy(w_h.at[pl.ds(r0 * _CHUNK, _GRP)], wb, isem),
      ]
      for cp in ils:
        cp.wait()
      half = _NCH // 2
      cps_a = [
          pltpu.async_copy(table_h.at[idxb.at[j]],
                           gb.at[pl.ds(j * _CHUNK, _CHUNK)], sem)
          for j in range(half)
      ]
      cps_b = [
          pltpu.async_copy(table_h.at[idxb.at[j]],
                           gb.at[pl.ds(j * _CHUNK, _CHUNK)], sem2)
          for j in range(half, _NCH)
      ]

      def scale(lo16, hi16):
        for k16 in range(lo16, hi16):
          wv16 = wb[pl.ds(k16 * 16, 16)]
          for j in range(16):
            e = k16 * 16 + j
            wv = wv16[j]
            gb[e, pl.ds(0, 16)] = gb[e, pl.ds(0, 16)] * wv
            gb[e, pl.ds(16, 16)] = gb[e, pl.ds(16, 16)] * wv

      def scatter(j):
        return pltpu.async_copy(gb.at[pl.ds(j * _CHUNK, _CHUNK)],
                                acc.at[dstb.at[j]], ssem, add=True)

      for cp in cps_a:
        cp.wait()
      scale(0, half * _CHUNK // 16)
      scs = [scatter(j) for j in range(half)]
      for cp in cps_b:
        cp.wait()
      scale(half * _CHUNK // 16, _GRP // 16)
      scs += [scatter(j) for j in range(half, _NCH)]
      for cp in scs:
        cp.wait()
      return carry

    lax.fori_loop(0, _GROUPS, grp, 0)
    plsc.subcore_barrier()
    pltpu.sync_copy(acc.at[slab], out_h.at[c, slab])

  return body(table, gidx, dst, w, zeros)


def _hyper_and_split(user_emb, item_emb, user_mat, item_mat):
  def body(u_ref, i_ref, mu_ref, mi_ref, hu_ref, hi_ref, us_ref, is_ref):
    u = u_ref[...]
    it = i_ref[...]
    hu_ref[...] = jnp.dot(u, mu_ref[...], preferred_element_type=jnp.float32)
    hi_ref[...] = jnp.dot(it, mi_ref[...], preferred_element_type=jnp.float32)
    us_ref[...] = jnp.stack([u[:, :32], u[:, 32:]], axis=0)
    is_ref[...] = jnp.stack([it[:, :32], it[:, 32:]], axis=0)

  grid = (_U // _BLK,)
  return pl.pallas_call(
      body,
      grid=grid,
      in_specs=[
          pl.BlockSpec((_BLK, _D), lambda i: (i, 0)),
          pl.BlockSpec((_BLK, _D), lambda i: (i, 0)),
          pl.BlockSpec((_D, _H), lambda i: (0, 0)),
          pl.BlockSpec((_D, _H), lambda i: (0, 0)),
      ],
      out_specs=[
          pl.BlockSpec((_BLK, _H), lambda i: (i, 0)),
          pl.BlockSpec((_BLK, _H), lambda i: (i, 0)),
          pl.BlockSpec((2, _BLK, 32), lambda i: (0, i, 0)),
          pl.BlockSpec((2, _BLK, 32), lambda i: (0, i, 0)),
      ],
      out_shape=[
          jax.ShapeDtypeStruct((_U, _H), jnp.float32),
          jax.ShapeDtypeStruct((_I, _H), jnp.float32),
          jax.ShapeDtypeStruct((2, _U, 32), jnp.float32),
          jax.ShapeDtypeStruct((2, _I, 32), jnp.float32),
      ],
  )(user_emb, item_emb, user_mat, item_mat)


def _he_reduce(hyper_u, hyper_i, u_split, i_split):
  def body(hu_ref, us_ref, hi_ref, is_ref, heu_ref, hei_ref):
    step = pl.program_id(0)
    u = jnp.concatenate([us_ref[0], us_ref[1]], axis=1)
    it = jnp.concatenate([is_ref[0], is_ref[1]], axis=1)
    dn = (((0,), (0,)), ((), ()))
    pu = lax.dot_general(hu_ref[...], u, dn,
                         preferred_element_type=jnp.float32)
    pi = lax.dot_general(hi_ref[...], it, dn,
                         preferred_element_type=jnp.float32)

    @pl.when(step == 0)
    def _():
      heu_ref[...] = pu
      hei_ref[...] = pi

    @pl.when(step != 0)
    def _():
      heu_ref[...] += pu
      hei_ref[...] += pi

  grid = (_U // _BLK,)
  return pl.pallas_call(
      body,
      grid=grid,
      in_specs=[
          pl.BlockSpec((_BLK, _H), lambda i: (i, 0)),
          pl.BlockSpec((2, _BLK, 32), lambda i: (0, i, 0)),
          pl.BlockSpec((_BLK, _H), lambda i: (i, 0)),
          pl.BlockSpec((2, _BLK, 32), lambda i: (0, i, 0)),
      ],
      out_specs=[
          pl.BlockSpec((_H, _D), lambda i: (0, 0)),
          pl.BlockSpec((_H, _D), lambda i: (0, 0)),
      ],
      out_shape=[
          jax.ShapeDtypeStruct((_H, _D), jnp.float32),
          jax.ShapeDtypeStruct((_H, _D), jnp.float32),
      ],
  )(hyper_u, u_split, hyper_i, i_split)


def _combine(hyper_u, he_u, raw_u, acc_u, hyper_i, he_i, raw_i, acc_i):
  def body(hu_ref, heu_ref, ru_ref, au_ref, hi_ref, hei_ref, ri_ref, ai_ref,
           lu_ref, gu_ref, nu_ref, aou_ref, li_ref, gi_ref, ni_ref, aoi_ref):
    def one(h_ref, he_ref, r_ref, a_ref, l_ref, g_ref, n_ref, ao_ref):
      g = jnp.dot(h_ref[...], he_ref[...],
                  preferred_element_type=jnp.float32)
      raw = jnp.concatenate([r_ref[0], r_ref[1]], axis=1)
      loc = jnp.where(raw >= 0, raw, 0.5 * raw)
      nxt = loc + g
      l_ref[...] = loc
      g_ref[...] = g
      n_ref[...] = jnp.stack([nxt[:, :32], nxt[:, 32:]], axis=0)
      ao_ref[...] = a_ref[...] + nxt

    one(hu_ref, heu_ref, ru_ref, au_ref, lu_ref, gu_ref, nu_ref, aou_ref)
    one(hi_ref, hei_ref, ri_ref, ai_ref, li_ref, gi_ref, ni_ref, aoi_ref)

  grid = (_U // _BLK,)
  row_spec = pl.BlockSpec((_BLK, _D), lambda i: (i, 0))
  split_spec = pl.BlockSpec((2, _BLK, 32), lambda i: (0, i, 0))
  he_spec = pl.BlockSpec((_H, _D), lambda i: (0, 0))
  h_spec = pl.BlockSpec((_BLK, _H), lambda i: (i, 0))
  row_shape = jax.ShapeDtypeStruct((_U, _D), jnp.float32)
  split_shape = jax.ShapeDtypeStruct((2, _U, 32), jnp.float32)
  return pl.pallas_call(
      body,
      grid=grid,
      in_specs=[h_spec, he_spec, split_spec, row_spec] * 2,
      out_specs=[row_spec, row_spec, split_spec, row_spec] * 2,
      out_shape=[row_shape, row_shape, split_shape, row_shape] * 2,
  )(hyper_u, he_u, raw_u, acc_u, hyper_i, he_i, raw_i, acc_i)


def kernel(user_emb, item_emb, user_mat, item_mat, edge_index, edge_weight):
  row = edge_index[0].astype(jnp.int32)
  col = edge_index[1].astype(jnp.int32)
  pad = _E_PAD - _E
  rowp = jnp.pad(row, (0, pad))
  colp = jnp.pad(col, (0, pad))
  wp = jnp.pad(edge_weight.astype(jnp.float32), (0, pad))
  gidx_u = jnp.stack([colp, colp + _I]).reshape(_NC, _CHUNK_ROWS, _CHUNK)
  gidx_i = jnp.stack([rowp, rowp + _U]).reshape(_NC, _CHUNK_ROWS, _CHUNK)
  dst_u = rowp.reshape(_CHUNK_ROWS, _CHUNK)
  dst_i = colp.reshape(_CHUNK_ROWS, _CHUNK)
  zeros = jnp.zeros((_N_ACC, 32), jnp.float32)

  hyper_u, hyper_i, u_split, i_split = _hyper_and_split(
      user_emb, item_emb, user_mat, item_mat)

  acc_u, acc_i = user_emb, item_emb
  locals_u, locals_i, globals_u, globals_i = [], [], [], []
  for _ in range(2):
    i_tab = i_split.reshape(_NC * _I, 32)
    u_tab = u_split.reshape(_NC * _U, 32)
    raw_u = _sc_segment_sum(i_tab, gidx_u, dst_u, wp, zeros)
    raw_i = _sc_segment_sum(u_tab, gidx_i, dst_i, wp, zeros)
    he_u, he_i = _he_reduce(hyper_u, hyper_i, u_split, i_split)
    (local_u, g_u, u_split, acc_u,
     local_i, g_i, i_split, acc_i) = _combine(
         hyper_u, he_u, raw_u, acc_u, hyper_i, he_i, raw_i, acc_i)
    locals_u.append(local_u)
    locals_i.append(local_i)
    globals_u.append(g_u)
    globals_i.append(g_i)

  return (acc_u, acc_i, *locals_u, *locals_i, *globals_u, *globals_i)
